# baseline (speedup 1.0000x reference)
import functools

import jax
import jax.numpy as jnp
from jax import lax
from jax.experimental import pallas as pl
from jax.experimental.pallas import tpu as pltpu
from jax.experimental.pallas import tpu_sc as plsc

_NC = 2
_NS = 16
_NW = _NC * _NS
_CH = 32


def kernel(x, pos_table):
    B, T, D = x.shape
    R = B * T
    rows_per_w = R // _NW
    nch = rows_per_w // _CH
    xf = x.reshape(R, D)

    mesh = plsc.VectorSubcoreMesh(core_axis_name="c", subcore_axis_name="s")

    @functools.partial(
        pl.kernel,
        out_type=jax.ShapeDtypeStruct((R, D), jnp.float32),
        mesh=mesh,
        scratch_types=[
            pltpu.VMEM((2, _CH, D), jnp.float32),
            pltpu.VMEM((2, _CH, D), jnp.float32),
            pltpu.SemaphoreType.DMA((2,)),
            pltpu.SemaphoreType.DMA((2,)),
            pltpu.SemaphoreType.DMA((2,)),
        ],
    )
    def sc_add(x_hbm, p_hbm, o_hbm, buf, pbuf, xsem, psem, osem):
        wid = lax.axis_index("s") * _NC + lax.axis_index("c")
        base = wid * rows_per_w
        pbase = lax.rem(base, T)

        def start_load(k):
            slot = k % 2
            pltpu.make_async_copy(
                x_hbm.at[pl.ds(base + k * _CH, _CH)],
                buf.at[slot],
                xsem.at[slot],
            ).start()
            pltpu.make_async_copy(
                p_hbm.at[pl.ds(pbase + k * _CH, _CH)],
                pbuf.at[slot],
                psem.at[slot],
            ).start()

        start_load(0)
        start_load(1)

        for k in range(nch):
            slot = k % 2
            if k >= 2:
                pltpu.make_async_copy(
                    buf.at[slot],
                    o_hbm.at[pl.ds(base + (k - 2) * _CH, _CH)],
                    osem.at[slot],
                ).wait()
            pltpu.make_async_copy(
                x_hbm.at[pl.ds(base + k * _CH, _CH)],
                buf.at[slot],
                xsem.at[slot],
            ).wait()
            pltpu.make_async_copy(
                p_hbm.at[pl.ds(pbase + k * _CH, _CH)],
                pbuf.at[slot],
                psem.at[slot],
            ).wait()

            def row_add(r, _):
                for c in range(D // 16):
                    sl = pl.ds(16 * c, 16)
                    buf[slot, r, sl] = buf[slot, r, sl] + pbuf[slot, r, sl]
                return _

            lax.fori_loop(0, _CH, row_add, 0)

            pltpu.make_async_copy(
                buf.at[slot],
                o_hbm.at[pl.ds(base + k * _CH, _CH)],
                osem.at[slot],
            ).start()
            if k + 2 < nch:
                start_load(k + 2)

        for k in range(max(nch - 2, 0), nch):
            slot = k % 2
            pltpu.make_async_copy(
                buf.at[slot],
                o_hbm.at[pl.ds(base + k * _CH, _CH)],
                osem.at[slot],
            ).wait()

    out = sc_add(xf, pos_table)
    return out.reshape(B, T, D)


# speedup vs baseline: 4.1484x; 4.1484x over previous
import jax
import jax.numpy as jnp
from jax.experimental import pallas as pl
from jax.experimental.pallas import tpu as pltpu


def _add_kernel(x_hbm, p_hbm, o_hbm, xbuf, obuf, pbuf, xsem, psem, osem):
    T = x_hbm.shape[1]
    H = T // 2

    for i in range(2):
        pltpu.make_async_copy(
            p_hbm.at[pl.ds(i * H, H)], pbuf.at[i], psem.at[i]
        ).start()
        pltpu.make_async_copy(
            x_hbm.at[:, pl.ds(i * H, H)], xbuf.at[i], xsem.at[i]
        ).start()

    for i in range(2):
        pltpu.make_async_copy(
            p_hbm.at[pl.ds(i * H, H)], pbuf.at[i], psem.at[i]
        ).wait()
        pltpu.make_async_copy(
            x_hbm.at[:, pl.ds(i * H, H)], xbuf.at[i], xsem.at[i]
        ).wait()
        obuf[i] = xbuf[i] + pbuf[i]
        pltpu.make_async_copy(
            obuf.at[i], o_hbm.at[:, pl.ds(i * H, H)], osem.at[i]
        ).start()

    for i in range(2):
        pltpu.make_async_copy(
            obuf.at[i], o_hbm.at[:, pl.ds(i * H, H)], osem.at[i]
        ).wait()


def kernel(x, pos_table):
    B, T, D = x.shape
    return pl.pallas_call(
        _add_kernel,
        in_specs=[
            pl.BlockSpec(memory_space=pl.ANY),
            pl.BlockSpec(memory_space=pl.ANY),
        ],
        out_specs=pl.BlockSpec(memory_space=pl.ANY),
        out_shape=jax.ShapeDtypeStruct((B, T, D), x.dtype),
        scratch_shapes=[
            pltpu.VMEM((2, B, T // 2, D), x.dtype),
            pltpu.VMEM((2, B, T // 2, D), x.dtype),
            pltpu.VMEM((2, T // 2, D), x.dtype),
            pltpu.SemaphoreType.DMA((2,)),
            pltpu.SemaphoreType.DMA((2,)),
            pltpu.SemaphoreType.DMA((2,)),
        ],
    )(x, pos_table)


# speedup vs baseline: 4.3879x; 1.0577x over previous
import jax
import jax.numpy as jnp
from jax.experimental import pallas as pl
from jax.experimental.pallas import tpu as pltpu


def _add_kernel(x_hbm, p_hbm, o_hbm, xbuf, obuf, pbuf, xsem, psem, osem):
    nb = x_hbm.shape[0] // 2

    pltpu.make_async_copy(p_hbm, pbuf, psem).start()
    for i in range(nb):
        pltpu.make_async_copy(
            x_hbm.at[pl.ds(2 * i, 2)], xbuf.at[i], xsem.at[i]
        ).start()
    pltpu.make_async_copy(p_hbm, pbuf, psem).wait()

    for i in range(nb):
        pltpu.make_async_copy(
            x_hbm.at[pl.ds(2 * i, 2)], xbuf.at[i], xsem.at[i]
        ).wait()
        obuf[i] = xbuf[i] + pbuf[...]
        pltpu.make_async_copy(
            obuf.at[i], o_hbm.at[pl.ds(2 * i, 2)], osem.at[i]
        ).start()

    for i in range(nb):
        pltpu.make_async_copy(
            obuf.at[i], o_hbm.at[pl.ds(2 * i, 2)], osem.at[i]
        ).wait()


def kernel(x, pos_table):
    B, T, D = x.shape
    return pl.pallas_call(
        _add_kernel,
        in_specs=[
            pl.BlockSpec(memory_space=pl.ANY),
            pl.BlockSpec(memory_space=pl.ANY),
        ],
        out_specs=pl.BlockSpec(memory_space=pl.ANY),
        out_shape=jax.ShapeDtypeStruct((B, T, D), x.dtype),
        scratch_shapes=[
            pltpu.VMEM((B // 2, 2, T, D), x.dtype),
            pltpu.VMEM((B // 2, 2, T, D), x.dtype),
            pltpu.VMEM((T, D), x.dtype),
            pltpu.SemaphoreType.DMA((B // 2,)),
            pltpu.SemaphoreType.DMA,
            pltpu.SemaphoreType.DMA((B // 2,)),
        ],
    )(x, pos_table)
